# baseline (speedup 1.0000x reference)
import functools

import jax
import jax.numpy as jnp
from jax import lax
from jax.experimental import pallas as pl
from jax.experimental.pallas import tpu as pltpu
from jax.experimental.pallas import tpu_sc as plsc

N = 50000
E = 800000
B = 64
HIDDEN = 128
G = 64
T = 3

NP = 51200
NPQ = NP // 4

_NC = 2
_NS = 16
_NW = _NC * _NS

_SC_MESH = dict(core_axis_name="c", subcore_axis_name="s")


_GCHUNK = 256
_GSUB = 128


def _gather_rows(table, idx, n_idx):
    per_w = n_idx // _NW
    n_full = per_w // _GCHUNK
    has_tail = per_w % _GCHUNK != 0

    mesh = plsc.VectorSubcoreMesh(**_SC_MESH)

    @functools.partial(
        pl.kernel,
        out_type=jax.ShapeDtypeStruct((n_idx, 2 * B), jnp.float32),
        mesh=mesh,
        scratch_types=[
            pltpu.VMEM((_GCHUNK,), jnp.int32),
            pltpu.VMEM((_GCHUNK, 2 * B), jnp.float32),
            pltpu.SemaphoreType.DMA,
        ],
    )
    def k(table_hbm, idx_hbm, out_hbm, idx_v, rows_v, sem):
        wid = lax.axis_index("s") * _NC + lax.axis_index("c")
        base_w = wid * per_w

        def do_chunk(off):
            pltpu.sync_copy(idx_hbm.at[pl.ds(off, _GCHUNK)], idx_v)
            descs = []
            for j in range(_GCHUNK // _GSUB):
                descs.append(pltpu.async_copy(
                    table_hbm.at[idx_v.at[pl.ds(j * _GSUB, _GSUB)]],
                    rows_v.at[pl.ds(j * _GSUB, _GSUB)], sem))
            for d in descs:
                d.wait()
            pltpu.sync_copy(rows_v, out_hbm.at[pl.ds(off, _GCHUNK)])

        def body(i, carry):
            do_chunk(base_w + i * _GCHUNK)
            return carry

        lax.fori_loop(0, n_full, body, 0)
        if has_tail:
            do_chunk(base_w + per_w - _GCHUNK)

    return k(table, idx)


_SCHUNK = 64
_NCH = E // _SCHUNK
_NCH_PAD = 12512
_CH_TILE = _NCH_PAD // _NS
EP = _NCH_PAD * _SCHUNK
_STCH = 80


def _scatter_add_c(m, sel, c_in):
    mesh = plsc.VectorSubcoreMesh(**_SC_MESH)

    @functools.partial(
        pl.kernel,
        out_type=jax.ShapeDtypeStruct((NP, 2 * B), jnp.float32),
        mesh=mesh,
        scratch_types=[
            pltpu.VMEM((_SCHUNK,), jnp.int32),
            pltpu.VMEM((_STCH, 2 * B), jnp.float32),
            pltpu.VMEM_SHARED((NPQ + 8, 2 * B), jnp.float32),
        ],
    )
    def k(m_hbm, sel_hbm, cin_hbm, cout_hbm, idx_v, rows_v, acc_sh):
        c = lax.axis_index("c")
        s = lax.axis_index("s")

        for p in range(2):
            qq = 2 * p + c
            node_base = qq * NPQ

            def stage_in(j, carry):
                r = (s + j * _NS) * _STCH
                pltpu.sync_copy(cin_hbm.at[pl.ds(node_base + r, _STCH)],
                                rows_v)
                pltpu.sync_copy(rows_v, acc_sh.at[pl.ds(r, _STCH)])
                return carry

            lax.fori_loop(0, 10, stage_in, 0)
            plsc.subcore_barrier()

            def body(i, carry):
                ch = s + i * _NS
                ch_m = jnp.minimum(ch, _NCH - 1)
                pltpu.sync_copy(
                    sel_hbm.at[pl.ds(qq * EP + ch * _SCHUNK, _SCHUNK)],
                    idx_v)
                pltpu.sync_copy(
                    m_hbm.at[pl.ds(ch_m * _SCHUNK, _SCHUNK)],
                    rows_v.at[pl.ds(0, _SCHUNK)])
                pltpu.sync_copy(rows_v.at[pl.ds(0, _SCHUNK)],
                                acc_sh.at[idx_v], add=True)
                return carry

            lax.fori_loop(0, _CH_TILE, body, 0)
            plsc.subcore_barrier()

            def stage_out(j, carry):
                r = (s + j * _NS) * _STCH
                pltpu.sync_copy(acc_sh.at[pl.ds(r, _STCH)], rows_v)
                pltpu.sync_copy(rows_v,
                                cout_hbm.at[pl.ds(node_base + r, _STCH)])
                return carry

            lax.fori_loop(0, 10, stage_out, 0)
            plsc.subcore_barrier()

    return k(m, sel, c_in)


def _selidx_kernel(d_ref, o_ref):
    d = d_ref[...]
    for qq in range(4):
        lo = qq * NPQ
        o_ref[qq] = jnp.where((d >= lo) & (d < lo + NPQ), d - lo, NPQ)


def _sel_indices(dst):
    d_pad = jnp.concatenate(
        [dst, jnp.full((EP - E,), NP, jnp.int32)]).reshape(EP // 128, 128)
    out = pl.pallas_call(
        _selidx_kernel,
        in_specs=[pl.BlockSpec(d_pad.shape, lambda: (0, 0))],
        out_specs=pl.BlockSpec((4,) + d_pad.shape, lambda: (0, 0, 0)),
        out_shape=jax.ShapeDtypeStruct((4,) + d_pad.shape, jnp.int32),
    )(d_pad)
    return out.reshape(4 * EP)


def _linear_kernel(x_ref, w_ref, b_ref, o_ref):
    x = x_ref[...][:, :B]
    o_ref[...] = lax.dot_general(
        x, w_ref[...], (((1,), (1,)), ((), ())),
        preferred_element_type=jnp.float32) + b_ref[...]


def _linear(x, w, b, blk):
    n = x.shape[0]
    return pl.pallas_call(
        _linear_kernel,
        grid=(n // blk,),
        in_specs=[
            pl.BlockSpec((blk, x.shape[1]), lambda i: (i, 0)),
            pl.BlockSpec(w.shape, lambda i: (0, 0)),
            pl.BlockSpec((1, b.shape[0]), lambda i: (0, 0)),
        ],
        out_specs=pl.BlockSpec((blk, w.shape[0]), lambda i: (i, 0)),
        out_shape=jax.ShapeDtypeStruct((n, w.shape[0]), jnp.float32),
    )(x, w, b.reshape(1, -1))


def _msg_kernel(g_ref, d_ref, w_ref, o_ref):
    cd = g_ref[:, :B] * d_ref[...]
    o_ref[...] = jnp.tanh(lax.dot_general(
        cd, w_ref[...], (((1,), (1,)), ((), ())),
        preferred_element_type=jnp.float32))


def _msg(g, d, fc_W128, blk=8000):
    return pl.pallas_call(
        _msg_kernel,
        grid=(E // blk,),
        in_specs=[
            pl.BlockSpec((blk, 2 * B), lambda i: (i, 0)),
            pl.BlockSpec((blk, B), lambda i: (i, 0)),
            pl.BlockSpec((2 * B, B), lambda i: (0, 0)),
        ],
        out_specs=pl.BlockSpec((blk, 2 * B), lambda i: (i, 0)),
        out_shape=jax.ShapeDtypeStruct((E, 2 * B), jnp.float32),
    )(g, d, fc_W128)


_RBLK = 400


def _readout_kernel(c_ref, batch_ref, w1_ref, b1_ref, w2_ref, b2_ref, o_ref):
    i = pl.program_id(0)
    h = jnp.tanh(lax.dot_general(
        c_ref[...][:, :B], w1_ref[...], (((1,), (1,)), ((), ())),
        preferred_element_type=jnp.float32) + b1_ref[...])
    out = lax.dot_general(
        h, w2_ref[...], (((1,), (1,)), ((), ())),
        preferred_element_type=jnp.float32) + b2_ref[...]
    seg = batch_ref[0, 0, :]
    onehot = (seg[:, None] == lax.broadcasted_iota(
        jnp.int32, (_RBLK, G), 1)).astype(jnp.float32)
    pooled = lax.dot_general(
        onehot, out, (((0,), (0,)), ((), ())),
        preferred_element_type=jnp.float32)

    @pl.when(i == 0)
    def _():
        o_ref[...] = jnp.zeros_like(o_ref)

    o_ref[...] += pooled


def _readout(c_pad, batch, w1, b1, w2, b2):
    batch3 = batch.astype(jnp.int32).reshape(N // _RBLK, 1, _RBLK)
    return pl.pallas_call(
        _readout_kernel,
        grid=(N // _RBLK,),
        in_specs=[
            pl.BlockSpec((_RBLK, 2 * B), lambda i: (i, 0)),
            pl.BlockSpec((1, 1, _RBLK), lambda i: (i, 0, 0)),
            pl.BlockSpec((HIDDEN, B), lambda i: (0, 0)),
            pl.BlockSpec((1, HIDDEN), lambda i: (0, 0)),
            pl.BlockSpec((4, HIDDEN), lambda i: (0, 0)),
            pl.BlockSpec((1, 4), lambda i: (0, 0)),
        ],
        out_specs=pl.BlockSpec((G, 4), lambda i: (0, 0)),
        out_shape=jax.ShapeDtypeStruct((G, 4), jnp.float32),
    )(c_pad, batch3, w1, b1.reshape(1, -1), w2, b2.reshape(1, -1))


def kernel(Z, edge_index, edge_attr, batch, embed, cf_W, cf_b, df_W, df_b,
           fc_W, mlp1_W, mlp1_b, mlp2_W, mlp2_b):
    src = edge_index[0].astype(jnp.int32)
    dst = edge_index[1].astype(jnp.int32)
    z_pad = jnp.concatenate(
        [Z.astype(jnp.int32), jnp.zeros((NP - N,), jnp.int32)])

    embed128 = jnp.pad(embed, ((0, 0), (0, B)))
    c = _gather_rows(embed128, z_pad, NP)

    d = _linear(edge_attr, df_W, df_b, blk=8000)

    cf_W128 = jnp.pad(cf_W, ((0, B), (0, 0)))
    cf_b128 = jnp.pad(cf_b, (0, B))
    fc_W128 = jnp.pad(fc_W, ((0, B), (0, 0)))

    sel = _sel_indices(dst)

    for _ in range(T):
        cc = _linear(c, cf_W128, cf_b128, blk=6400)
        g = _gather_rows(cc, src, E)
        m = _msg(g, d, fc_W128)
        c = _scatter_add_c(m, sel, c)

    return _readout(c, batch, mlp1_W, mlp1_b, mlp2_W, mlp2_b)
